# baseline (speedup 1.0000x reference)
import functools

import jax
import jax.numpy as jnp
from jax import lax
from jax.experimental import pallas as pl
from jax.experimental.pallas import tpu as pltpu
from jax.experimental.pallas import tpu_sc as plsc

NC = 2
NS = 16
LANES = 16
BATCH = 128
NQ = 4
DQ = 32


def _addsplit_body(x_ref, e_ref, o_ref):
    res = x_ref[...] + e_ref[...]
    for q in range(NQ):
        o_ref[q] = res[:, q * DQ:(q + 1) * DQ]


def _embed_addsplit(x, node_emb, bm, n_t):
    n, d_in = x.shape
    return pl.pallas_call(
        _addsplit_body,
        grid=(n // bm,),
        in_specs=[
            pl.BlockSpec((bm, d_in), lambda i: (i, 0)),
            pl.BlockSpec((bm, d_in), lambda i: (i, 0)),
        ],
        out_specs=pl.BlockSpec((NQ, bm, DQ), lambda i: (0, i, 0)),
        out_shape=jax.ShapeDtypeStruct((NQ, n_t, DQ), jnp.float32),
    )(x, node_emb)


def _finalize_body(agg_ref, deg_ref, w_ref, o_ref):
    a = jnp.concatenate([agg_ref[q] for q in range(NQ)], axis=1)
    d = deg_ref[0, :, 0:1] + deg_ref[1, :, 0:1]
    m = jnp.dot(a, w_ref[...], preferred_element_type=jnp.float32)
    o_ref[...] = jnp.tanh(m / jnp.maximum(d, 1.0))


NB = 8
LOOK = 4


def _sc_agg_body(xe_hbm, src_hbm, dst_hbm, zeros_hbm, zeros16_hbm, ones16_hbm,
                 agg_hbm, deg_hbm,
                 src_v, dst_v, rows_v, ones16_v,
                 table_sp, agg_sp, deg_sp,
                 gsem0, gsem1, gsem2, gsem3, gsem4, gsem5, gsem6, gsem7,
                 ssem0, ssem1, ssem2, ssem3, ssem4, ssem5, ssem6, ssem7,
                 *, nbatch, rows_pt, chunk):
    cid = lax.axis_index("c")
    sid = lax.axis_index("s")
    rows = [rows_v.at[pl.ds(b * BATCH, BATCH)] for b in range(NB)]
    gsems = (gsem0, gsem1, gsem2, gsem3, gsem4, gsem5, gsem6, gsem7)
    ssems = (ssem0, ssem1, ssem2, ssem3, ssem4, ssem5, ssem6, ssem7)

    pltpu.sync_copy(src_hbm.at[sid], src_v)
    pltpu.sync_copy(dst_hbm.at[sid], dst_v)
    pltpu.sync_copy(ones16_hbm, ones16_v)

    rbase = sid * rows_pt
    tbase = sid * chunk
    ngrp = nbatch // NB
    half = ngrp // 2

    for p in range(2):
        q = 2 * p + cid

        pltpu.sync_copy(xe_hbm.at[q, pl.ds(tbase, chunk)],
                        table_sp.at[pl.ds(tbase, chunk)])
        pltpu.sync_copy(zeros_hbm, agg_sp.at[pl.ds(rbase, rows_pt)])
        if p == 0:
            pltpu.sync_copy(zeros16_hbm, deg_sp.at[pl.ds(rbase, rows_pt)])

        plsc.subcore_barrier()

        for b in range(LOOK):
            pltpu.async_copy(table_sp.at[src_v.at[b]], rows[b], gsems[b])

        def ebody(g, c):
            my_deg = (g < half) == (cid == 0)
            for b in range(NB):
                j = NB * g + b
                bn = (b + LOOK) % NB
                pltpu.make_async_copy(table_sp.at[src_v.at[j]], rows[b],
                                      gsems[b]).wait()
                pltpu.async_copy(rows[b], agg_sp.at[dst_v.at[j]], ssems[b],
                                 add=True)

                if p == 0:
                    @pl.when(my_deg)
                    def _():
                        pltpu.sync_copy(ones16_v, deg_sp.at[dst_v.at[j]],
                                        add=True)

                @pl.when(j + LOOK < nbatch)
                def _():
                    @pl.when(j + LOOK >= NB)
                    def _():
                        pltpu.make_async_copy(
                            rows[bn], agg_sp.at[dst_v.at[j + LOOK - NB]],
                            ssems[bn]).wait()
                    pltpu.async_copy(table_sp.at[src_v.at[j + LOOK]],
                                     rows[bn], gsems[bn])
            return c

        lax.fori_loop(0, ngrp, ebody, 0)

        for j in range(max(0, nbatch - NB), nbatch):
            b = j % NB
            pltpu.make_async_copy(rows[b], agg_sp.at[dst_v.at[j]],
                                  ssems[b]).wait()

        plsc.subcore_barrier()

        pltpu.sync_copy(agg_sp.at[pl.ds(rbase, rows_pt)],
                        agg_hbm.at[q, pl.ds(rbase, rows_pt)])
        if p == 0:
            pltpu.sync_copy(deg_sp.at[pl.ds(rbase, rows_pt)],
                            deg_hbm.at[cid, pl.ds(rbase, rows_pt)])


def kernel(x, edge_index, W, node_emb):
    n, d_in = x.shape
    e = edge_index.shape[1]

    per_tile = -(-e // NS)
    nbatch = NB * (-(-per_tile // (NB * BATCH)))
    e_pad = NS * nbatch * BATCH

    n_pad = -(-(n + 1) // (NS * 64)) * (NS * 64)
    rows_pt = n_pad // NS
    n_t = -(-n // NS) * NS
    chunk = n_t // NS

    src = edge_index[0]
    dst = edge_index[1]
    pad = e_pad - e
    src_p = jnp.concatenate(
        [src, jnp.zeros((pad,), jnp.int32)]).reshape(NS, nbatch, BATCH)
    dst_p = jnp.concatenate(
        [dst, jnp.full((pad,), n, jnp.int32)]).reshape(NS, nbatch, BATCH)

    bm = next((b for b in (1024, 1000, 512, 500, 256, 250, 128, 125, 64,
                           40, 16, 8) if n % b == 0), n)
    xe4 = _embed_addsplit(x, node_emb, bm, n_t)

    mesh = plsc.VectorSubcoreMesh(core_axis_name="c", subcore_axis_name="s",
                                  num_cores=NC, num_subcores=NS)
    body = functools.partial(_sc_agg_body, nbatch=nbatch, rows_pt=rows_pt,
                             chunk=chunk)
    zeros_h = jnp.zeros((rows_pt, DQ), jnp.float32)
    zeros16_h = jnp.zeros((rows_pt, 16), jnp.float32)
    ones16_h = jnp.ones((BATCH, 16), jnp.float32)
    agg, deg = pl.kernel(
        body,
        out_type=[
            jax.ShapeDtypeStruct((NQ, n_pad, DQ), jnp.float32),
            jax.ShapeDtypeStruct((NC, n_pad, 16), jnp.float32),
        ],
        mesh=mesh,
        compiler_params=pltpu.CompilerParams(use_tc_tiling_on_sc=False),
        scratch_types=[
            pltpu.VMEM((nbatch, BATCH), jnp.int32),
            pltpu.VMEM((nbatch, BATCH), jnp.int32),
            pltpu.VMEM((NB * BATCH, DQ), jnp.float32),
            pltpu.VMEM((BATCH, 16), jnp.float32),
            pltpu.VMEM_SHARED((n_t, DQ), jnp.float32),
            pltpu.VMEM_SHARED((n_pad, DQ), jnp.float32),
            pltpu.VMEM_SHARED((n_pad, 16), jnp.float32),
        ] + [pltpu.SemaphoreType.DMA] * (2 * NB),
    )(xe4, src_p, dst_p, zeros_h, zeros16_h, ones16_h)

    bm2 = next((b for b in (1000, 500, 250, 125, 200, 100, 50, 25, 8)
                if n % b == 0), n)
    out = pl.pallas_call(
        _finalize_body,
        grid=(n // bm2,),
        in_specs=[
            pl.BlockSpec((NQ, bm2, DQ), lambda i: (0, i, 0)),
            pl.BlockSpec((NC, bm2, 16), lambda i: (0, i, 0)),
            pl.BlockSpec((d_in, d_in), lambda i: (0, 0)),
        ],
        out_specs=pl.BlockSpec((bm2, d_in), lambda i: (i, 0)),
        out_shape=jax.ShapeDtypeStruct((n, d_in), jnp.float32),
    )(agg, deg, W)

    return out
